# speedup vs baseline: 1.5765x; 1.5765x over previous
import functools

import jax
import jax.numpy as jnp
from jax import lax
from jax.experimental import pallas as pl
from jax.experimental.pallas import tpu as pltpu
from jax.experimental.pallas import tpu_sc as plsc

D_MODEL = 1024
NUM_WORKERS = 32
TOTAL = 4 * 4096
B_PER_W = TOTAL // NUM_WORKERS
CHUNK = 64
NCHUNKS = B_PER_W // CHUNK

_mesh = plsc.VectorSubcoreMesh(core_axis_name="c", subcore_axis_name="s")


@functools.partial(
    pl.kernel,
    out_type=jax.ShapeDtypeStruct((TOTAL, D_MODEL), jnp.float32),
    mesh=_mesh,
    scratch_types=[
        pltpu.VMEM((NCHUNKS, CHUNK), jnp.int32),
        pltpu.VMEM((CHUNK, D_MODEL), jnp.float32),
        pltpu.SemaphoreType.DMA,
    ],
)
def _embed_sc(x_hbm, table_hbm, out_hbm, idx_v, rows_v, gsem):
    wid = lax.axis_index("s") * 2 + lax.axis_index("c")
    base = wid * B_PER_W
    pltpu.sync_copy(x_hbm.at[wid], idx_v)

    def body(g, carry):
        pltpu.async_copy(table_hbm.at[idx_v.at[g]], rows_v, gsem).wait()
        pltpu.sync_copy(rows_v, out_hbm.at[pl.ds(base + g * CHUNK, CHUNK)])
        return carry

    lax.fori_loop(0, NCHUNKS, body, 0)


def kernel(x, table):
    x3 = x.reshape(NUM_WORKERS, NCHUNKS, CHUNK).astype(jnp.int32)
    out = _embed_sc(x3, table)
    return out.reshape(x.shape[0], x.shape[1], D_MODEL)


# speedup vs baseline: 1.6237x; 1.0299x over previous
import functools

import jax
import jax.numpy as jnp
from jax import lax
from jax.experimental import pallas as pl
from jax.experimental.pallas import tpu as pltpu
from jax.experimental.pallas import tpu_sc as plsc

D_MODEL = 1024
NUM_WORKERS = 32
TOTAL = 4 * 4096
B_PER_W = TOTAL // NUM_WORKERS
CHUNK = 32
NCHUNKS = B_PER_W // CHUNK

_mesh = plsc.VectorSubcoreMesh(core_axis_name="c", subcore_axis_name="s")


@functools.partial(
    pl.kernel,
    out_type=jax.ShapeDtypeStruct((TOTAL, D_MODEL), jnp.float32),
    mesh=_mesh,
    scratch_types=[
        pltpu.VMEM((NCHUNKS, CHUNK), jnp.int32),
        pltpu.VMEM((CHUNK, D_MODEL), jnp.float32),
        pltpu.VMEM((CHUNK, D_MODEL), jnp.float32),
        pltpu.SemaphoreType.DMA,
        pltpu.SemaphoreType.DMA,
        pltpu.SemaphoreType.DMA,
        pltpu.SemaphoreType.DMA,
    ],
)
def _embed_sc(x_hbm, table_hbm, out_hbm, idx_v, rows_a, rows_b,
              gsem_a, gsem_b, ssem_a, ssem_b):
    wid = lax.axis_index("s") * 2 + lax.axis_index("c")
    base = wid * B_PER_W
    pltpu.sync_copy(x_hbm.at[wid], idx_v)

    bufs = (rows_a, rows_b)
    gsems = (gsem_a, gsem_b)
    ssems = (ssem_a, ssem_b)

    gathers = [None] * NCHUNKS
    scatters = [None] * NCHUNKS
    gathers[0] = pltpu.async_copy(
        table_hbm.at[idx_v.at[0]], bufs[0], gsems[0])
    for g in range(NCHUNKS):
        cur = g % 2
        nxt = 1 - cur
        if g + 1 < NCHUNKS:
            if g >= 1:
                scatters[g - 1].wait()
            gathers[g + 1] = pltpu.async_copy(
                table_hbm.at[idx_v.at[g + 1]], bufs[nxt], gsems[nxt])
        gathers[g].wait()
        scatters[g] = pltpu.async_copy(
            bufs[cur], out_hbm.at[pl.ds(base + g * CHUNK, CHUNK)], ssems[cur])
    scatters[NCHUNKS - 2].wait()
    scatters[NCHUNKS - 1].wait()


def kernel(x, table):
    x3 = x.reshape(NUM_WORKERS, NCHUNKS, CHUNK).astype(jnp.int32)
    out = _embed_sc(x3, table)
    return out.reshape(x.shape[0], x.shape[1], D_MODEL)


# speedup vs baseline: 1.6402x; 1.0102x over previous
import functools

import jax
import jax.numpy as jnp
from jax import lax
from jax.experimental import pallas as pl
from jax.experimental.pallas import tpu as pltpu
from jax.experimental.pallas import tpu_sc as plsc

D_MODEL = 1024
NUM_WORKERS = 32
TOTAL = 4 * 4096
B_PER_W = TOTAL // NUM_WORKERS
CHUNK = 32
NCHUNKS = B_PER_W // CHUNK

_mesh = plsc.VectorSubcoreMesh(core_axis_name="c", subcore_axis_name="s")


@functools.partial(
    pl.kernel,
    out_type=jax.ShapeDtypeStruct((TOTAL, D_MODEL), jnp.float32),
    mesh=_mesh,
    scratch_types=[
        pltpu.VMEM((NCHUNKS, CHUNK), jnp.int32),
        pltpu.VMEM((CHUNK, D_MODEL), jnp.float32),
        pltpu.VMEM((CHUNK, D_MODEL), jnp.float32),
        pltpu.VMEM((CHUNK, D_MODEL), jnp.float32),
        pltpu.SemaphoreType.DMA,
        pltpu.SemaphoreType.DMA,
        pltpu.SemaphoreType.DMA,
        pltpu.SemaphoreType.DMA,
        pltpu.SemaphoreType.DMA,
        pltpu.SemaphoreType.DMA,
    ],
)
def _embed_sc(x_hbm, table_hbm, out_hbm, idx_v, rows_a, rows_b, rows_c,
              gsem_a, gsem_b, gsem_c, ssem_a, ssem_b, ssem_c):
    wid = lax.axis_index("s") * 2 + lax.axis_index("c")
    base = wid * B_PER_W
    pltpu.sync_copy(x_hbm.at[wid], idx_v)

    NBUF = 3
    bufs = (rows_a, rows_b, rows_c)
    gsems = (gsem_a, gsem_b, gsem_c)
    ssems = (ssem_a, ssem_b, ssem_c)

    gathers = [None] * NCHUNKS
    scatters = [None] * NCHUNKS
    for g in range(NBUF):
        gathers[g] = pltpu.async_copy(
            table_hbm.at[idx_v.at[g]], bufs[g], gsems[g])
    for g in range(NCHUNKS):
        cur = g % NBUF
        gathers[g].wait()
        scatters[g] = pltpu.async_copy(
            bufs[cur], out_hbm.at[pl.ds(base + g * CHUNK, CHUNK)], ssems[cur])
        if g + NBUF < NCHUNKS:
            scatters[g].wait()
            gathers[g + NBUF] = pltpu.async_copy(
                table_hbm.at[idx_v.at[g + NBUF]], bufs[cur], gsems[cur])
    for g in range(NCHUNKS - NBUF, NCHUNKS):
        scatters[g].wait()


def kernel(x, table):
    x3 = x.reshape(NUM_WORKERS, NCHUNKS, CHUNK).astype(jnp.int32)
    out = _embed_sc(x3, table)
    return out.reshape(x.shape[0], x.shape[1], D_MODEL)


# speedup vs baseline: 1.6581x; 1.0109x over previous
import functools

import jax
import jax.numpy as jnp
from jax import lax
from jax.experimental import pallas as pl
from jax.experimental.pallas import tpu as pltpu
from jax.experimental.pallas import tpu_sc as plsc

D_MODEL = 1024
NUM_WORKERS = 32
TOTAL = 4 * 4096
B_PER_W = TOTAL // NUM_WORKERS
CHUNK = 16
NCHUNKS = B_PER_W // CHUNK

_mesh = plsc.VectorSubcoreMesh(core_axis_name="c", subcore_axis_name="s")


@functools.partial(
    pl.kernel,
    out_type=jax.ShapeDtypeStruct((TOTAL, D_MODEL), jnp.float32),
    mesh=_mesh,
    scratch_types=[
        pltpu.VMEM((NCHUNKS, CHUNK), jnp.int32),
        pltpu.VMEM((CHUNK, D_MODEL), jnp.float32),
        pltpu.VMEM((CHUNK, D_MODEL), jnp.float32),
        pltpu.VMEM((CHUNK, D_MODEL), jnp.float32),
        pltpu.VMEM((CHUNK, D_MODEL), jnp.float32),
        pltpu.VMEM((CHUNK, D_MODEL), jnp.float32),
        pltpu.VMEM((CHUNK, D_MODEL), jnp.float32),
        pltpu.SemaphoreType.DMA,
        pltpu.SemaphoreType.DMA,
        pltpu.SemaphoreType.DMA,
        pltpu.SemaphoreType.DMA,
        pltpu.SemaphoreType.DMA,
        pltpu.SemaphoreType.DMA,
        pltpu.SemaphoreType.DMA,
        pltpu.SemaphoreType.DMA,
        pltpu.SemaphoreType.DMA,
        pltpu.SemaphoreType.DMA,
        pltpu.SemaphoreType.DMA,
        pltpu.SemaphoreType.DMA,
    ],
)
def _embed_sc(x_hbm, table_hbm, out_hbm, idx_v,
              rows_a, rows_b, rows_c, rows_d, rows_e, rows_f,
              gsem_a, gsem_b, gsem_c, gsem_d, gsem_e, gsem_f,
              ssem_a, ssem_b, ssem_c, ssem_d, ssem_e, ssem_f):
    wid = lax.axis_index("s") * 2 + lax.axis_index("c")
    base = wid * B_PER_W
    pltpu.sync_copy(x_hbm.at[wid], idx_v)

    NBUF = 6
    bufs = (rows_a, rows_b, rows_c, rows_d, rows_e, rows_f)
    gsems = (gsem_a, gsem_b, gsem_c, gsem_d, gsem_e, gsem_f)
    ssems = (ssem_a, ssem_b, ssem_c, ssem_d, ssem_e, ssem_f)

    gathers = [None] * NCHUNKS
    scatters = [None] * NCHUNKS
    for g in range(NBUF):
        gathers[g] = pltpu.async_copy(
            table_hbm.at[idx_v.at[g]], bufs[g], gsems[g])
    for g in range(NCHUNKS):
        cur = g % NBUF
        gathers[g].wait()
        scatters[g] = pltpu.async_copy(
            bufs[cur], out_hbm.at[pl.ds(base + g * CHUNK, CHUNK)], ssems[cur])
        if g + NBUF < NCHUNKS:
            scatters[g].wait()
            gathers[g + NBUF] = pltpu.async_copy(
                table_hbm.at[idx_v.at[g + NBUF]], bufs[cur], gsems[cur])
    for g in range(NCHUNKS - NBUF, NCHUNKS):
        scatters[g].wait()


def kernel(x, table):
    x3 = x.reshape(NUM_WORKERS, NCHUNKS, CHUNK).astype(jnp.int32)
    out = _embed_sc(x3, table)
    return out.reshape(x.shape[0], x.shape[1], D_MODEL)


# speedup vs baseline: 1.6694x; 1.0068x over previous
import functools

import jax
import jax.numpy as jnp
from jax import lax
from jax.experimental import pallas as pl
from jax.experimental.pallas import tpu as pltpu
from jax.experimental.pallas import tpu_sc as plsc

D_MODEL = 1024
NUM_WORKERS = 32
TOTAL = 4 * 4096
B_PER_W = TOTAL // NUM_WORKERS
CHUNK = 16
NCHUNKS = B_PER_W // CHUNK

_mesh = plsc.VectorSubcoreMesh(core_axis_name="c", subcore_axis_name="s")


@functools.partial(
    pl.kernel,
    out_type=jax.ShapeDtypeStruct((4, 4096, D_MODEL), jnp.float32),
    mesh=_mesh,
    scratch_types=[
        pltpu.VMEM((B_PER_W,), jnp.int32),
        pltpu.VMEM((CHUNK, D_MODEL), jnp.float32),
        pltpu.VMEM((CHUNK, D_MODEL), jnp.float32),
        pltpu.VMEM((CHUNK, D_MODEL), jnp.float32),
        pltpu.VMEM((CHUNK, D_MODEL), jnp.float32),
        pltpu.VMEM((CHUNK, D_MODEL), jnp.float32),
        pltpu.VMEM((CHUNK, D_MODEL), jnp.float32),
        pltpu.SemaphoreType.DMA,
        pltpu.SemaphoreType.DMA,
        pltpu.SemaphoreType.DMA,
        pltpu.SemaphoreType.DMA,
        pltpu.SemaphoreType.DMA,
        pltpu.SemaphoreType.DMA,
        pltpu.SemaphoreType.DMA,
        pltpu.SemaphoreType.DMA,
        pltpu.SemaphoreType.DMA,
        pltpu.SemaphoreType.DMA,
        pltpu.SemaphoreType.DMA,
        pltpu.SemaphoreType.DMA,
    ],
)
def _embed_sc(x_hbm, table_hbm, out_hbm, idx_v,
              rows_a, rows_b, rows_c, rows_d, rows_e, rows_f,
              gsem_a, gsem_b, gsem_c, gsem_d, gsem_e, gsem_f,
              ssem_a, ssem_b, ssem_c, ssem_d, ssem_e, ssem_f):
    wid = lax.axis_index("s") * 2 + lax.axis_index("c")
    row = wid // 8
    col0 = (wid % 8) * B_PER_W
    pltpu.sync_copy(x_hbm.at[row, pl.ds(col0, B_PER_W)], idx_v)

    NBUF = 6
    bufs = (rows_a, rows_b, rows_c, rows_d, rows_e, rows_f)
    gsems = (gsem_a, gsem_b, gsem_c, gsem_d, gsem_e, gsem_f)
    ssems = (ssem_a, ssem_b, ssem_c, ssem_d, ssem_e, ssem_f)

    gathers = [None] * NCHUNKS
    scatters = [None] * NCHUNKS
    for g in range(NBUF):
        gathers[g] = pltpu.async_copy(
            table_hbm.at[idx_v.at[pl.ds(g * CHUNK, CHUNK)]], bufs[g], gsems[g])
    for g in range(NCHUNKS):
        cur = g % NBUF
        gathers[g].wait()
        scatters[g] = pltpu.async_copy(
            bufs[cur],
            out_hbm.at[row, pl.ds(col0 + g * CHUNK, CHUNK)], ssems[cur])
        if g + NBUF < NCHUNKS:
            scatters[g].wait()
            gathers[g + NBUF] = pltpu.async_copy(
                table_hbm.at[idx_v.at[pl.ds((g + NBUF) * CHUNK, CHUNK)]],
                bufs[cur], gsems[cur])
    for g in range(NCHUNKS - NBUF, NCHUNKS):
        scatters[g].wait()


def kernel(x, table):
    return _embed_sc(x.astype(jnp.int32), table)
